# Initial kernel scaffold; baseline (speedup 1.0000x reference)
import functools

import jax
import jax.numpy as jnp
from jax import lax
from jax.experimental import pallas as pl
from jax.experimental.pallas import tpu as pltpu
from jax.experimental.pallas import tpu_sc as plsc

N_NODES = 100000
D_FEAT = 128
N_COMM = 5
BATCH = 500000

C_PAD = 16
ROW_BLK = 2000

NC, NS = 2, 16
NW = NC * NS
B_PAD = 524288
B_PER_W = B_PAD // NW
CHUNK = 2048
N_CHUNK = B_PER_W // CHUNK


def _mlp_body(nf_ref, w1_ref, b1_ref, w2_ref, b2_ref, out_ref):
    x = nf_ref[...]
    h = jnp.dot(x, w1_ref[...], preferred_element_type=jnp.float32)
    h = jnp.maximum(h + b1_ref[...], 0.0)
    logits = jnp.dot(h, w2_ref[...], preferred_element_type=jnp.float32)
    logits = logits + b2_ref[...]
    m = jnp.max(logits, axis=-1, keepdims=True)
    e = jnp.exp(logits - m)
    out_ref[...] = e / jnp.sum(e, axis=-1, keepdims=True)


_mlp_call = pl.pallas_call(
    _mlp_body,
    grid=(N_NODES // ROW_BLK,),
    in_specs=[
        pl.BlockSpec((ROW_BLK, D_FEAT), lambda i: (i, 0)),
        pl.BlockSpec((D_FEAT, D_FEAT), lambda i: (0, 0)),
        pl.BlockSpec((1, D_FEAT), lambda i: (0, 0)),
        pl.BlockSpec((D_FEAT, C_PAD), lambda i: (0, 0)),
        pl.BlockSpec((1, C_PAD), lambda i: (0, 0)),
    ],
    out_specs=pl.BlockSpec((ROW_BLK, C_PAD), lambda i: (i, 0)),
    out_shape=jax.ShapeDtypeStruct((N_NODES, C_PAD), jnp.float32),
)

_sc_mesh = plsc.VectorSubcoreMesh(
    core_axis_name="c", subcore_axis_name="s", num_cores=NC, num_subcores=NS
)


@functools.partial(
    pl.kernel,
    out_type=jax.ShapeDtypeStruct((B_PAD, C_PAD), jnp.float32),
    mesh=_sc_mesh,
    scratch_types=[
        pltpu.VMEM((CHUNK,), jnp.int32),
        pltpu.VMEM((CHUNK, C_PAD), jnp.float32),
        pltpu.SemaphoreType.DMA,
    ],
)
def _sc_gather(table_hbm, idx_hbm, out_hbm, idx_v, rows_v, sem):
    wid = lax.axis_index("s") * NC + lax.axis_index("c")
    base = wid * B_PER_W

    @pl.loop(0, N_CHUNK)
    def _chunk(i):
        off = base + i * CHUNK
        pltpu.sync_copy(idx_hbm.at[pl.ds(off, CHUNK)], idx_v)
        pltpu.async_copy(table_hbm.at[idx_v], rows_v, sem).wait()
        pltpu.sync_copy(rows_v, out_hbm.at[pl.ds(off, CHUNK)])


def kernel(node_features, nodes, W1, b1, W2, b2):
    w2p = jnp.zeros((D_FEAT, C_PAD), jnp.float32).at[:, :N_COMM].set(W2)
    b2p = jnp.full((C_PAD,), -1e30, jnp.float32).at[:N_COMM].set(b2)
    table = _mlp_call(
        node_features, W1, b1.reshape(1, D_FEAT), w2p, b2p.reshape(1, C_PAD)
    )
    nodes_p = jnp.zeros((B_PAD,), jnp.int32).at[:BATCH].set(nodes)
    out = _sc_gather(table, nodes_p)
    return out[:BATCH, :N_COMM]


# speedup vs baseline: 1.8222x; 1.8222x over previous
import functools

import jax
import jax.numpy as jnp
from jax import lax
from jax.experimental import pallas as pl
from jax.experimental.pallas import tpu as pltpu
from jax.experimental.pallas import tpu_sc as plsc

N_NODES = 100000
D_FEAT = 128
N_COMM = 5
BATCH = 500000

C_PAD = 16
ROW_BLK = 2000

NC, NS = 2, 16
NW = NC * NS
B_PAD = 524288
B_PER_W = B_PAD // NW
CHUNK = 2048
N_CHUNK = B_PER_W // CHUNK


def _mlp_body(nf_ref, w1_ref, b1_ref, w2_ref, b2_ref, out_ref):
    x = nf_ref[...]
    h = jnp.dot(x, w1_ref[...], preferred_element_type=jnp.float32)
    h = jnp.maximum(h + b1_ref[...], 0.0)
    logits = jnp.dot(h, w2_ref[...], preferred_element_type=jnp.float32)
    logits = logits + b2_ref[...]
    m = jnp.max(logits, axis=-1, keepdims=True)
    e = jnp.exp(logits - m)
    out_ref[...] = e / jnp.sum(e, axis=-1, keepdims=True)


_mlp_call = pl.pallas_call(
    _mlp_body,
    grid=(N_NODES // ROW_BLK,),
    in_specs=[
        pl.BlockSpec((ROW_BLK, D_FEAT), lambda i: (i, 0)),
        pl.BlockSpec((D_FEAT, D_FEAT), lambda i: (0, 0)),
        pl.BlockSpec((1, D_FEAT), lambda i: (0, 0)),
        pl.BlockSpec((D_FEAT, C_PAD), lambda i: (0, 0)),
        pl.BlockSpec((1, C_PAD), lambda i: (0, 0)),
    ],
    out_specs=pl.BlockSpec((ROW_BLK, C_PAD), lambda i: (i, 0)),
    out_shape=jax.ShapeDtypeStruct((N_NODES, C_PAD), jnp.float32),
)

_sc_mesh = plsc.VectorSubcoreMesh(
    core_axis_name="c", subcore_axis_name="s", num_cores=NC, num_subcores=NS
)


@functools.partial(
    pl.kernel,
    out_type=jax.ShapeDtypeStruct((B_PAD, C_PAD), jnp.float32),
    mesh=_sc_mesh,
    scratch_types=[
        pltpu.VMEM((CHUNK,), jnp.int32),
        pltpu.VMEM((CHUNK, C_PAD), jnp.float32),
        pltpu.SemaphoreType.DMA,
    ],
    compiler_params=pltpu.CompilerParams(use_tc_tiling_on_sc=False),
)
def _sc_gather(table_hbm, idx_hbm, out_hbm, idx_v, rows_v, sem):
    wid = lax.axis_index("s") * NC + lax.axis_index("c")
    base = wid * B_PER_W

    @pl.loop(0, N_CHUNK)
    def _chunk(i):
        off = base + i * CHUNK
        pltpu.sync_copy(idx_hbm.at[pl.ds(off, CHUNK)], idx_v)
        pltpu.async_copy(table_hbm.at[idx_v], rows_v, sem).wait()
        pltpu.sync_copy(rows_v, out_hbm.at[pl.ds(off, CHUNK)])


def kernel(node_features, nodes, W1, b1, W2, b2):
    w2p = jnp.zeros((D_FEAT, C_PAD), jnp.float32).at[:, :N_COMM].set(W2)
    b2p = jnp.full((C_PAD,), -1e30, jnp.float32).at[:N_COMM].set(b2)
    table = _mlp_call(
        node_features, W1, b1.reshape(1, D_FEAT), w2p, b2p.reshape(1, C_PAD)
    )
    nodes_p = jnp.zeros((B_PAD,), jnp.int32).at[:BATCH].set(nodes)
    out = _sc_gather(table, nodes_p)
    return out[:BATCH, :N_COMM]


# speedup vs baseline: 1.8557x; 1.0184x over previous
import functools

import jax
import jax.numpy as jnp
from jax import lax
from jax.experimental import pallas as pl
from jax.experimental.pallas import tpu as pltpu
from jax.experimental.pallas import tpu_sc as plsc

N_NODES = 100000
D_FEAT = 128
N_COMM = 5
BATCH = 500000

C_PAD = 8
ROW_BLK = 2000

NC, NS = 2, 16
NW = NC * NS
B_PAD = 524288
B_PER_W = B_PAD // NW
CHUNK = 2048
N_CHUNK = B_PER_W // CHUNK
NBUF = 4


def _mlp_body(nf_ref, w1_ref, b1_ref, w2_ref, b2_ref, out_ref):
    x = nf_ref[...]
    h = jnp.dot(x, w1_ref[...], preferred_element_type=jnp.float32)
    h = jnp.maximum(h + b1_ref[...], 0.0)
    logits = jnp.dot(h, w2_ref[...], preferred_element_type=jnp.float32)
    logits = logits + b2_ref[...]
    m = jnp.max(logits, axis=-1, keepdims=True)
    e = jnp.exp(logits - m)
    out_ref[...] = e / jnp.sum(e, axis=-1, keepdims=True)


_mlp_call = pl.pallas_call(
    _mlp_body,
    grid=(N_NODES // ROW_BLK,),
    in_specs=[
        pl.BlockSpec((ROW_BLK, D_FEAT), lambda i: (i, 0)),
        pl.BlockSpec((D_FEAT, D_FEAT), lambda i: (0, 0)),
        pl.BlockSpec((1, D_FEAT), lambda i: (0, 0)),
        pl.BlockSpec((D_FEAT, C_PAD), lambda i: (0, 0)),
        pl.BlockSpec((1, C_PAD), lambda i: (0, 0)),
    ],
    out_specs=pl.BlockSpec((ROW_BLK, C_PAD), lambda i: (i, 0)),
    out_shape=jax.ShapeDtypeStruct((N_NODES, C_PAD), jnp.float32),
)

_sc_mesh = plsc.VectorSubcoreMesh(
    core_axis_name="c", subcore_axis_name="s", num_cores=NC, num_subcores=NS
)


@functools.partial(
    pl.kernel,
    out_type=jax.ShapeDtypeStruct((B_PAD, C_PAD), jnp.float32),
    mesh=_sc_mesh,
    scratch_types=[
        [pltpu.VMEM((CHUNK,), jnp.int32)] * NBUF,
        [pltpu.VMEM((CHUNK, C_PAD), jnp.float32)] * NBUF,
        [pltpu.SemaphoreType.DMA] * NBUF,
        [pltpu.SemaphoreType.DMA] * NBUF,
    ],
    compiler_params=pltpu.CompilerParams(use_tc_tiling_on_sc=False),
)
def _sc_gather(table_hbm, idx_hbm, out_hbm, idx_vs, rows_vs, gsems, osems):
    wid = lax.axis_index("s") * NC + lax.axis_index("c")
    base = wid * B_PER_W

    gd = [None] * N_CHUNK
    od = [None] * N_CHUNK
    for i in range(N_CHUNK):
        b = i % NBUF
        if i >= NBUF:
            od[i - NBUF].wait()
        off = base + i * CHUNK
        pltpu.sync_copy(idx_hbm.at[pl.ds(off, CHUNK)], idx_vs[b])
        gd[i] = pltpu.async_copy(table_hbm.at[idx_vs[b]], rows_vs[b], gsems[b])
        if i >= 1:
            p = i - 1
            gd[p].wait()
            od[p] = pltpu.async_copy(
                rows_vs[p % NBUF], out_hbm.at[pl.ds(base + p * CHUNK, CHUNK)],
                osems[p % NBUF],
            )
    last = N_CHUNK - 1
    gd[last].wait()
    od[last] = pltpu.async_copy(
        rows_vs[last % NBUF], out_hbm.at[pl.ds(base + last * CHUNK, CHUNK)],
        osems[last % NBUF],
    )
    for i in range(max(0, N_CHUNK - NBUF), N_CHUNK):
        od[i].wait()


def kernel(node_features, nodes, W1, b1, W2, b2):
    w2p = jnp.zeros((D_FEAT, C_PAD), jnp.float32).at[:, :N_COMM].set(W2)
    b2p = jnp.full((C_PAD,), -1e30, jnp.float32).at[:N_COMM].set(b2)
    table = _mlp_call(
        node_features, W1, b1.reshape(1, D_FEAT), w2p, b2p.reshape(1, C_PAD)
    )
    nodes_p = jnp.zeros((B_PAD,), jnp.int32).at[:BATCH].set(nodes)
    out = _sc_gather(table, nodes_p)
    return out[:BATCH, :N_COMM]


# speedup vs baseline: 3.6213x; 1.9514x over previous
import functools

import jax
import jax.numpy as jnp
from jax import lax
from jax.experimental import pallas as pl
from jax.experimental.pallas import tpu as pltpu
from jax.experimental.pallas import tpu_sc as plsc

N_NODES = 100000
D_FEAT = 128
N_COMM = 5
BATCH = 500000

C_PAD = 8
ROW_BLK = 2000

NC, NS = 2, 16
NW = NC * NS
B_PAD = 524288
B_PER_W = B_PAD // NW
CHUNK = 2048
N_CHUNK = B_PER_W // CHUNK
NBUF = 3
TPC = CHUNK // 128

FMT_TILES = 32
FMT_BLK = FMT_TILES * 128
FMT_GRID = -(-BATCH // FMT_BLK)


def _mlp_body(nf_ref, w1_ref, b1_ref, w2_ref, b2_ref, out_ref):
    x = nf_ref[...]
    h = jnp.dot(x, w1_ref[...], preferred_element_type=jnp.float32)
    h = jnp.maximum(h + b1_ref[...], 0.0)
    logits = jnp.dot(h, w2_ref[...], preferred_element_type=jnp.float32)
    logits = logits + b2_ref[...]
    m = jnp.max(logits, axis=-1, keepdims=True)
    e = jnp.exp(logits - m)
    out_ref[...] = e / jnp.sum(e, axis=-1, keepdims=True)


_mlp_call = pl.pallas_call(
    _mlp_body,
    grid=(N_NODES // ROW_BLK,),
    in_specs=[
        pl.BlockSpec((ROW_BLK, D_FEAT), lambda i: (i, 0)),
        pl.BlockSpec((D_FEAT, D_FEAT), lambda i: (0, 0)),
        pl.BlockSpec((1, D_FEAT), lambda i: (0, 0)),
        pl.BlockSpec((D_FEAT, C_PAD), lambda i: (0, 0)),
        pl.BlockSpec((1, C_PAD), lambda i: (0, 0)),
    ],
    out_specs=pl.BlockSpec((ROW_BLK, C_PAD), lambda i: (i, 0)),
    out_shape=jax.ShapeDtypeStruct((N_NODES, C_PAD), jnp.float32),
)

_sc_mesh = plsc.VectorSubcoreMesh(
    core_axis_name="c", subcore_axis_name="s", num_cores=NC, num_subcores=NS
)


@functools.partial(
    pl.kernel,
    out_type=jax.ShapeDtypeStruct((B_PAD // 128, C_PAD, 128), jnp.float32),
    mesh=_sc_mesh,
    scratch_types=[
        [pltpu.VMEM((CHUNK,), jnp.int32)] * NBUF,
        [pltpu.VMEM((CHUNK, C_PAD), jnp.float32)] * NBUF,
        [pltpu.VMEM((TPC, C_PAD, 128), jnp.float32)] * NBUF,
        [pltpu.SemaphoreType.DMA] * NBUF,
        [pltpu.SemaphoreType.DMA] * NBUF,
    ],
    compiler_params=pltpu.CompilerParams(
        use_tc_tiling_on_sc=False, needs_layout_passes=False
    ),
)
def _sc_gather(table_hbm, idx_hbm, out_hbm, idx_vs, rows_vs, xt_vs, gsems, osems):
    wid = lax.axis_index("s") * NC + lax.axis_index("c")
    base = wid * B_PER_W
    lane = lax.iota(jnp.int32, 16)

    def _transpose(b):
        rows = rows_vs[b]
        xt = xt_vs[b]

        @pl.loop(0, TPC)
        def _tile(t):
            r0 = t * 128
            for c in range(C_PAD):
                cvec = jnp.full((16,), c, jnp.int32)
                for g in range(8):
                    ridx = r0 + g * 16 + lane
                    xt[t, c, pl.ds(g * 16, 16)] = plsc.load_gather(
                        rows, [ridx, cvec]
                    )

    def _out_copy(i):
        b = i % NBUF
        return pltpu.async_copy(
            xt_vs[b],
            out_hbm.at[pl.ds((base + i * CHUNK) // 128, TPC)],
            osems[b],
        )

    gd = [None] * N_CHUNK
    od = [None] * N_CHUNK
    for i in range(N_CHUNK):
        b = i % NBUF
        if i >= NBUF:
            od[i - NBUF].wait()
        off = base + i * CHUNK
        pltpu.sync_copy(idx_hbm.at[pl.ds(off, CHUNK)], idx_vs[b])
        gd[i] = pltpu.async_copy(table_hbm.at[idx_vs[b]], rows_vs[b], gsems[b])
        if i >= 1:
            gd[i - 1].wait()
            _transpose((i - 1) % NBUF)
            od[i - 1] = _out_copy(i - 1)
    gd[N_CHUNK - 1].wait()
    _transpose((N_CHUNK - 1) % NBUF)
    od[N_CHUNK - 1] = _out_copy(N_CHUNK - 1)
    for i in range(max(0, N_CHUNK - NBUF), N_CHUNK):
        od[i].wait()


def _fmt_body(x_ref, out_ref):
    for j in range(FMT_TILES):
        out_ref[:, j * 128:(j + 1) * 128] = x_ref[j, :N_COMM, :]


_fmt_call = pl.pallas_call(
    _fmt_body,
    grid=(FMT_GRID,),
    in_specs=[pl.BlockSpec((FMT_TILES, C_PAD, 128), lambda i: (i, 0, 0))],
    out_specs=pl.BlockSpec((N_COMM, FMT_BLK), lambda i: (0, i)),
    out_shape=jax.ShapeDtypeStruct((N_COMM, BATCH), jnp.float32),
)


def kernel(node_features, nodes, W1, b1, W2, b2):
    w2p = jnp.zeros((D_FEAT, C_PAD), jnp.float32).at[:, :N_COMM].set(W2)
    b2p = jnp.full((C_PAD,), -1e30, jnp.float32).at[:N_COMM].set(b2)
    table = _mlp_call(
        node_features, W1, b1.reshape(1, D_FEAT), w2p, b2p.reshape(1, C_PAD)
    )
    nodes_p = jnp.zeros((B_PAD,), jnp.int32).at[:BATCH].set(nodes)
    tiles = _sc_gather(table, nodes_p)
    out_t = _fmt_call(tiles)
    return out_t.T


# speedup vs baseline: 4.1603x; 1.1489x over previous
import functools

import jax
import jax.numpy as jnp
from jax import lax
from jax.experimental import pallas as pl
from jax.experimental.pallas import tpu as pltpu
from jax.experimental.pallas import tpu_sc as plsc

N_NODES = 100000
D_FEAT = 128
N_COMM = 5
BATCH = 500000

C_PAD = 8
ROW_BLK = 2000

NC, NS = 2, 16
NW = NC * NS
B_PAD = 524288
B_PER_W = B_PAD // NW
CHUNK = 2048
N_CHUNK = B_PER_W // CHUNK
NBUF = 2
ROW_W = 16
TPC = CHUNK // 128

FMT_TILES = 64
FMT_BLK = FMT_TILES * 128
FMT_GRID = -(-BATCH // FMT_BLK)


def _mlp_body(nf_ref, w1_ref, b1_ref, w2_ref, b2_ref, out_ref):
    x = nf_ref[...]
    h = jnp.dot(x, w1_ref[...], preferred_element_type=jnp.float32)
    h = jnp.maximum(h + b1_ref[...], 0.0)
    logits = jnp.dot(h, w2_ref[...], preferred_element_type=jnp.float32)
    logits = logits + b2_ref[...]
    m = jnp.max(logits, axis=-1, keepdims=True)
    e = jnp.exp(logits - m)
    p = e / jnp.sum(e, axis=-1, keepdims=True)
    out_ref[:, :, 0:C_PAD] = p.reshape(ROW_BLK // 8, 8, C_PAD)


_mlp_call = pl.pallas_call(
    _mlp_body,
    grid=(N_NODES // ROW_BLK,),
    in_specs=[
        pl.BlockSpec((ROW_BLK, D_FEAT), lambda i: (i, 0)),
        pl.BlockSpec((D_FEAT, D_FEAT), lambda i: (0, 0)),
        pl.BlockSpec((1, D_FEAT), lambda i: (0, 0)),
        pl.BlockSpec((D_FEAT, C_PAD), lambda i: (0, 0)),
        pl.BlockSpec((1, C_PAD), lambda i: (0, 0)),
    ],
    out_specs=pl.BlockSpec((ROW_BLK // 8, 8, 128), lambda i: (i, 0, 0)),
    out_shape=jax.ShapeDtypeStruct((N_NODES // 8, 8, 128), jnp.float32),
)

_sc_mesh = plsc.VectorSubcoreMesh(
    core_axis_name="c", subcore_axis_name="s", num_cores=NC, num_subcores=NS
)


@functools.partial(
    pl.kernel,
    out_type=jax.ShapeDtypeStruct((B_PAD // 128, C_PAD, 128), jnp.float32),
    mesh=_sc_mesh,
    scratch_types=[
        [pltpu.VMEM((CHUNK,), jnp.int32)] * NBUF,
        [pltpu.VMEM((CHUNK, ROW_W), jnp.float32)] * NBUF,
        [pltpu.VMEM((TPC, C_PAD, 128), jnp.float32)] * NBUF,
        [pltpu.SemaphoreType.DMA] * NBUF,
        [pltpu.SemaphoreType.DMA] * NBUF,
    ],
    compiler_params=pltpu.CompilerParams(
        use_tc_tiling_on_sc=False, needs_layout_passes=False
    ),
)
def _sc_gather(table_hbm, idx_hbm, out_hbm, idx_vs, rows_vs, xt_vs, gsems, osems):
    wid = lax.axis_index("s") * NC + lax.axis_index("c")
    base = wid * B_PER_W
    lane = lax.iota(jnp.int32, 16)

    def _transpose(b):
        rows = rows_vs[b]
        xt = xt_vs[b]

        @pl.loop(0, TPC)
        def _tile(t):
            r0 = t * 128
            for c in range(C_PAD):
                cvec = jnp.full((16,), c, jnp.int32)
                for g in range(8):
                    ridx = r0 + g * 16 + lane
                    xt[t, c, pl.ds(g * 16, 16)] = plsc.load_gather(
                        rows, [ridx, cvec]
                    )

    def _out_copy(i):
        b = i % NBUF
        return pltpu.async_copy(
            xt_vs[b],
            out_hbm.at[pl.ds((base + i * CHUNK) // 128, TPC)],
            osems[b],
        )

    gd = [None] * N_CHUNK
    od = [None] * N_CHUNK
    for i in range(N_CHUNK):
        b = i % NBUF
        if i >= NBUF:
            od[i - NBUF].wait()
        off = base + i * CHUNK
        pltpu.sync_copy(idx_hbm.at[pl.ds(off, CHUNK)], idx_vs[b])
        gd[i] = pltpu.async_copy(table_hbm.at[idx_vs[b]], rows_vs[b], gsems[b])
        if i >= 1:
            gd[i - 1].wait()
            _transpose((i - 1) % NBUF)
            od[i - 1] = _out_copy(i - 1)
    gd[N_CHUNK - 1].wait()
    _transpose((N_CHUNK - 1) % NBUF)
    od[N_CHUNK - 1] = _out_copy(N_CHUNK - 1)
    for i in range(max(0, N_CHUNK - NBUF), N_CHUNK):
        od[i].wait()


def _fmt_body(x_ref, out_ref):
    for j in range(FMT_TILES):
        out_ref[:, j * 128:(j + 1) * 128] = x_ref[j, :N_COMM, :]


_fmt_call = pl.pallas_call(
    _fmt_body,
    grid=(FMT_GRID,),
    in_specs=[pl.BlockSpec((FMT_TILES, C_PAD, 128), lambda i: (i, 0, 0))],
    out_specs=pl.BlockSpec((N_COMM, FMT_BLK), lambda i: (0, i)),
    out_shape=jax.ShapeDtypeStruct((N_COMM, BATCH), jnp.float32),
)


def kernel(node_features, nodes, W1, b1, W2, b2):
    w2p = jnp.zeros((D_FEAT, C_PAD), jnp.float32).at[:, :N_COMM].set(W2)
    b2p = jnp.full((C_PAD,), -1e30, jnp.float32).at[:N_COMM].set(b2)
    table3 = _mlp_call(
        node_features, W1, b1.reshape(1, D_FEAT), w2p, b2p.reshape(1, C_PAD)
    )
    table = table3.reshape(N_NODES * 8, ROW_W)
    nodes_p = jnp.zeros((B_PAD,), jnp.int32).at[:BATCH].set(nodes * 8)
    tiles = _sc_gather(table, nodes_p)
    out_t = _fmt_call(tiles)
    return out_t.T


# speedup vs baseline: 4.6660x; 1.1216x over previous
import functools

import jax
import jax.numpy as jnp
from jax import lax
from jax.experimental import pallas as pl
from jax.experimental.pallas import tpu as pltpu
from jax.experimental.pallas import tpu_sc as plsc

N_NODES = 100000
D_FEAT = 128
N_COMM = 5
BATCH = 500000

C_PAD = 8
ROW_BLK = 4000

NC, NS = 2, 16
NW = NC * NS
B_PAD = 524288
B_PER_W = B_PAD // NW
CHUNK = 2048
N_CHUNK = B_PER_W // CHUNK
NBUF = 2
ROW_W = 16
TPC = CHUNK // 128

FMT_TILES = 128
FMT_BLK = FMT_TILES * 128
FMT_GRID = -(-BATCH // FMT_BLK)


def _mlp_body(nf_ref, w1_ref, b1_ref, w2_ref, b2_ref, out_ref):
    x = nf_ref[...]
    h = jnp.dot(x, w1_ref[...], preferred_element_type=jnp.float32)
    h = jnp.maximum(h + b1_ref[...], 0.0)
    logits = jnp.dot(h, w2_ref[...], preferred_element_type=jnp.float32)
    logits = logits + b2_ref[...]
    m = jnp.max(logits, axis=-1, keepdims=True)
    e = jnp.exp(logits - m)
    p = e / jnp.sum(e, axis=-1, keepdims=True)
    out_ref[:, :, 0:C_PAD] = p.reshape(ROW_BLK // 8, 8, C_PAD)


_mlp_call = pl.pallas_call(
    _mlp_body,
    grid=(N_NODES // ROW_BLK,),
    in_specs=[
        pl.BlockSpec((ROW_BLK, D_FEAT), lambda i: (i, 0)),
        pl.BlockSpec((D_FEAT, D_FEAT), lambda i: (0, 0)),
        pl.BlockSpec((1, D_FEAT), lambda i: (0, 0)),
        pl.BlockSpec((D_FEAT, C_PAD), lambda i: (0, 0)),
        pl.BlockSpec((1, C_PAD), lambda i: (0, 0)),
    ],
    out_specs=pl.BlockSpec((ROW_BLK // 8, 8, 128), lambda i: (i, 0, 0)),
    out_shape=jax.ShapeDtypeStruct((N_NODES // 8, 8, 128), jnp.float32),
)

_sc_mesh = plsc.VectorSubcoreMesh(
    core_axis_name="c", subcore_axis_name="s", num_cores=NC, num_subcores=NS
)


@functools.partial(
    pl.kernel,
    out_type=jax.ShapeDtypeStruct((B_PAD // 128, C_PAD, 128), jnp.float32),
    mesh=_sc_mesh,
    scratch_types=[
        [pltpu.VMEM((CHUNK,), jnp.int32)] * NBUF,
        [pltpu.VMEM((CHUNK, ROW_W), jnp.float32)] * NBUF,
        [pltpu.VMEM((TPC, C_PAD, 128), jnp.float32)] * NBUF,
        [pltpu.SemaphoreType.DMA] * NBUF,
        [pltpu.SemaphoreType.DMA] * NBUF,
    ],
    compiler_params=pltpu.CompilerParams(
        use_tc_tiling_on_sc=False, needs_layout_passes=False
    ),
)
def _sc_gather(table_hbm, idx_hbm, out_hbm, idx_vs, rows_vs, xt_vs, gsems, osems):
    wid = lax.axis_index("s") * NC + lax.axis_index("c")
    base = wid * B_PER_W
    lane = lax.iota(jnp.int32, 16)

    def _transpose(b):
        rows = rows_vs[b]
        xt = xt_vs[b]

        @pl.loop(0, TPC)
        def _tile(t):
            r0 = t * 128
            for c in range(C_PAD):
                cvec = jnp.full((16,), c, jnp.int32)
                for g in range(8):
                    ridx = r0 + g * 16 + lane
                    xt[t, c, pl.ds(g * 16, 16)] = plsc.load_gather(
                        rows, [ridx, cvec]
                    )

    def _out_copy(i):
        b = i % NBUF
        return pltpu.async_copy(
            xt_vs[b],
            out_hbm.at[pl.ds((base + i * CHUNK) // 128, TPC)],
            osems[b],
        )

    gd = [None] * N_CHUNK
    od = [None] * N_CHUNK
    for i in range(N_CHUNK):
        b = i % NBUF
        if i >= NBUF:
            od[i - NBUF].wait()
        off = base + i * CHUNK
        pltpu.sync_copy(idx_hbm.at[pl.ds(off, CHUNK)], idx_vs[b])
        gd[i] = pltpu.async_copy(table_hbm.at[idx_vs[b]], rows_vs[b], gsems[b])
        if i >= 1:
            gd[i - 1].wait()
            _transpose((i - 1) % NBUF)
            od[i - 1] = _out_copy(i - 1)
    gd[N_CHUNK - 1].wait()
    _transpose((N_CHUNK - 1) % NBUF)
    od[N_CHUNK - 1] = _out_copy(N_CHUNK - 1)
    for i in range(max(0, N_CHUNK - NBUF), N_CHUNK):
        od[i].wait()


def _fmt_body(x_ref, out_ref):
    for j in range(FMT_TILES):
        out_ref[:, j * 128:(j + 1) * 128] = x_ref[j, :N_COMM, :]


_fmt_call = pl.pallas_call(
    _fmt_body,
    grid=(FMT_GRID,),
    in_specs=[pl.BlockSpec((FMT_TILES, C_PAD, 128), lambda i: (i, 0, 0))],
    out_specs=pl.BlockSpec((N_COMM, FMT_BLK), lambda i: (0, i)),
    out_shape=jax.ShapeDtypeStruct((N_COMM, BATCH), jnp.float32),
)


def kernel(node_features, nodes, W1, b1, W2, b2):
    w2p = jnp.zeros((D_FEAT, C_PAD), jnp.float32).at[:, :N_COMM].set(W2)
    b2p = jnp.full((C_PAD,), -1e30, jnp.float32).at[:N_COMM].set(b2)
    table3 = _mlp_call(
        node_features, W1, b1.reshape(1, D_FEAT), w2p, b2p.reshape(1, C_PAD)
    )
    table = table3.reshape(N_NODES * 8, ROW_W)
    nodes_p = jnp.zeros((B_PAD,), jnp.int32).at[:BATCH].set(nodes * 8)
    tiles = _sc_gather(table, nodes_p)
    out_t = _fmt_call(tiles)
    return out_t.T


# speedup vs baseline: 6.9748x; 1.4948x over previous
import functools

import jax
import jax.numpy as jnp
from jax import lax
from jax.experimental import pallas as pl
from jax.experimental.pallas import tpu as pltpu
from jax.experimental.pallas import tpu_sc as plsc

N_NODES = 100000
D_FEAT = 128
N_COMM = 5
BATCH = 500000

C_PAD = 8
ROW_BLK = 5000

NC, NS = 2, 16
NW = NC * NS
TILES_TOT = -(-BATCH // 128)
B_EFF = TILES_TOT * 128
NT_BASE = TILES_TOT // NW
NT_XTRA = TILES_TOT % NW
CHUNK = 2048
NBUF = 2
ROW_W = 16
CHUNK_PLAN = [(k * CHUNK, CHUNK, CHUNK // 128) for k in range(7)]
CHUNK_PLAN.append((7 * CHUNK, NT_BASE * 128 - 7 * CHUNK, NT_BASE - 7 * 16))
XTRA_OFF = NT_BASE * 128

FMT_TILES = 128
FMT_BLK = FMT_TILES * 128
FMT_GRID = -(-BATCH // FMT_BLK)


def _mlp_body(nf_ref, w1_ref, b1_ref, w2_ref, b2_ref, out_ref):
    x = nf_ref[...]
    h = jnp.dot(x, w1_ref[...], preferred_element_type=jnp.float32)
    h = jnp.maximum(h + b1_ref[...], 0.0)
    logits = jnp.dot(h, w2_ref[...], preferred_element_type=jnp.float32)
    logits = logits + b2_ref[...]
    m = jnp.max(logits, axis=-1, keepdims=True)
    e = jnp.exp(logits - m)
    p = e / jnp.sum(e, axis=-1, keepdims=True)
    out_ref[:, :, 0:C_PAD] = p.reshape(ROW_BLK // 8, 8, C_PAD)


_mlp_call = pl.pallas_call(
    _mlp_body,
    grid=(N_NODES // ROW_BLK,),
    in_specs=[
        pl.BlockSpec((ROW_BLK, D_FEAT), lambda i: (i, 0)),
        pl.BlockSpec((D_FEAT, D_FEAT), lambda i: (0, 0)),
        pl.BlockSpec((1, D_FEAT), lambda i: (0, 0)),
        pl.BlockSpec((D_FEAT, C_PAD), lambda i: (0, 0)),
        pl.BlockSpec((1, C_PAD), lambda i: (0, 0)),
    ],
    out_specs=pl.BlockSpec((ROW_BLK // 8, 8, 128), lambda i: (i, 0, 0)),
    out_shape=jax.ShapeDtypeStruct((N_NODES // 8, 8, 128), jnp.float32),
)

_sc_mesh = plsc.VectorSubcoreMesh(
    core_axis_name="c", subcore_axis_name="s", num_cores=NC, num_subcores=NS
)


@functools.partial(
    pl.kernel,
    out_type=jax.ShapeDtypeStruct((TILES_TOT, C_PAD, 128), jnp.float32),
    mesh=_sc_mesh,
    scratch_types=[
        [pltpu.VMEM((CHUNK,), jnp.int32)] * NBUF,
        [pltpu.VMEM((CHUNK, ROW_W), jnp.float32)] * NBUF,
        [pltpu.VMEM((CHUNK // 128, C_PAD, 128), jnp.float32)] * NBUF,
        [pltpu.SemaphoreType.DMA] * NBUF,
        [pltpu.SemaphoreType.DMA] * NBUF,
    ],
    compiler_params=pltpu.CompilerParams(
        use_tc_tiling_on_sc=False, needs_layout_passes=False
    ),
)
def _sc_gather(table_hbm, idx_hbm, out_hbm, idx_vs, rows_vs, xt_vs, gsems, osems):
    wid = lax.axis_index("s") * NC + lax.axis_index("c")
    tile_base = wid * NT_BASE + jnp.minimum(wid, NT_XTRA)
    base = tile_base * 128
    lane = lax.iota(jnp.int32, 16)

    def _transpose(b, ntiles):
        rows = rows_vs[b]
        xt = xt_vs[b]

        @pl.loop(0, ntiles)
        def _tile(t):
            r0 = t * 128
            for c in range(C_PAD):
                cvec = jnp.full((16,), c, jnp.int32)
                for g in range(8):
                    ridx = r0 + g * 16 + lane
                    xt[t, c, pl.ds(g * 16, 16)] = plsc.load_gather(
                        rows, [ridx, cvec]
                    )

    def _gather_chunk(b, roff, n):
        pltpu.sync_copy(
            idx_hbm.at[pl.ds(base + roff, n)], idx_vs[b].at[pl.ds(0, n)]
        )
        return pltpu.async_copy(
            table_hbm.at[idx_vs[b].at[pl.ds(0, n)]],
            rows_vs[b].at[pl.ds(0, n)],
            gsems[b],
        )

    def _out_copy(i):
        roff, n, nt = CHUNK_PLAN[i]
        b = i % NBUF
        return pltpu.async_copy(
            xt_vs[b].at[pl.ds(0, nt)],
            out_hbm.at[pl.ds(tile_base + roff // 128, nt)],
            osems[b],
        )

    n_chunk = len(CHUNK_PLAN)
    gd = [None] * n_chunk
    od = [None] * n_chunk
    for i in range(n_chunk):
        b = i % NBUF
        if i >= NBUF:
            od[i - NBUF].wait()
        roff, n, nt = CHUNK_PLAN[i]
        gd[i] = _gather_chunk(b, roff, n)
        if i >= 1:
            gd[i - 1].wait()
            _transpose((i - 1) % NBUF, CHUNK_PLAN[i - 1][2])
            od[i - 1] = _out_copy(i - 1)
    gd[n_chunk - 1].wait()
    _transpose((n_chunk - 1) % NBUF, CHUNK_PLAN[n_chunk - 1][2])
    od[n_chunk - 1] = _out_copy(n_chunk - 1)
    for i in range(max(0, n_chunk - NBUF), n_chunk):
        od[i].wait()

    @pl.when(wid < NT_XTRA)
    def _extra():
        pltpu.sync_copy(
            idx_hbm.at[pl.ds(base + XTRA_OFF, 128)], idx_vs[0].at[pl.ds(0, 128)]
        )
        pltpu.async_copy(
            table_hbm.at[idx_vs[0].at[pl.ds(0, 128)]],
            rows_vs[0].at[pl.ds(0, 128)],
            gsems[0],
        ).wait()
        _transpose(0, 1)
        pltpu.sync_copy(
            xt_vs[0].at[pl.ds(0, 1)],
            out_hbm.at[pl.ds(tile_base + NT_BASE, 1)],
        )


def _fmt_body(x_ref, out_ref):
    for j in range(FMT_TILES):
        out_ref[:, j * 128:(j + 1) * 128] = x_ref[j, :N_COMM, :]


_fmt_call = pl.pallas_call(
    _fmt_body,
    grid=(FMT_GRID,),
    in_specs=[pl.BlockSpec((FMT_TILES, C_PAD, 128), lambda i: (i, 0, 0))],
    out_specs=pl.BlockSpec((N_COMM, FMT_BLK), lambda i: (0, i)),
    out_shape=jax.ShapeDtypeStruct((N_COMM, BATCH), jnp.float32),
)


def kernel(node_features, nodes, W1, b1, W2, b2):
    w2p = jnp.zeros((D_FEAT, C_PAD), jnp.float32).at[:, :N_COMM].set(W2)
    b2p = jnp.full((C_PAD,), -1e30, jnp.float32).at[:N_COMM].set(b2)
    table3 = _mlp_call(
        node_features, W1, b1.reshape(1, D_FEAT), w2p, b2p.reshape(1, C_PAD)
    )
    table = table3.reshape(N_NODES * 8, ROW_W)
    nodes_p = jnp.zeros((B_EFF,), jnp.int32).at[:BATCH].set(nodes * 8)
    tiles = _sc_gather(table, nodes_p)
    out_t = _fmt_call(tiles)
    return out_t.T


# speedup vs baseline: 8.1794x; 1.1727x over previous
import functools

import jax
import jax.numpy as jnp
from jax import lax
from jax.experimental import pallas as pl
from jax.experimental.pallas import tpu as pltpu
from jax.experimental.pallas import tpu_sc as plsc

N_NODES = 100000
D_FEAT = 128
N_COMM = 5
BATCH = 500000

C_PAD = 8
ROW_BLK = 5000

NC, NS = 2, 16
NW = NC * NS
TILES_TOT = -(-BATCH // 128)
B_EFF = TILES_TOT * 128
NT_BASE = TILES_TOT // NW
NT_XTRA = TILES_TOT % NW
CHUNK = 2048
NBUF = 2
ROW_W = 16
CHUNK_PLAN = [(k * CHUNK, CHUNK, CHUNK // 128) for k in range(7)]
CHUNK_PLAN.append((7 * CHUNK, NT_BASE * 128 - 7 * CHUNK, NT_BASE - 7 * 16))
XTRA_OFF = NT_BASE * 128

FMT_TILES = 128
FMT_BLK = FMT_TILES * 128
FMT_GRID = -(-BATCH // FMT_BLK)


def _mlp_body(nf_ref, w1_ref, b1_ref, w2_ref, b2_ref, out_ref):
    x = nf_ref[...]
    h = jnp.dot(x, w1_ref[...], preferred_element_type=jnp.float32)
    h = jnp.maximum(h + b1_ref[...], 0.0)
    logits = jnp.dot(h, w2_ref[...], preferred_element_type=jnp.float32)
    logits = logits + b2_ref[...]
    m = jnp.max(logits, axis=-1, keepdims=True)
    e = jnp.exp(logits - m)
    p = e / jnp.sum(e, axis=-1, keepdims=True)
    out_ref[:, :, 0:N_COMM] = p.reshape(ROW_BLK // 8, 8, N_COMM)


_mlp_call = pl.pallas_call(
    _mlp_body,
    grid=(N_NODES // ROW_BLK,),
    in_specs=[
        pl.BlockSpec((ROW_BLK, D_FEAT), lambda i: (i, 0)),
        pl.BlockSpec((D_FEAT, D_FEAT), lambda i: (0, 0)),
        pl.BlockSpec((1, D_FEAT), lambda i: (0, 0)),
        pl.BlockSpec((D_FEAT, N_COMM), lambda i: (0, 0)),
        pl.BlockSpec((1, N_COMM), lambda i: (0, 0)),
    ],
    out_specs=pl.BlockSpec((ROW_BLK // 8, 8, 128), lambda i: (i, 0, 0)),
    out_shape=jax.ShapeDtypeStruct((N_NODES // 8, 8, 128), jnp.float32),
)

_sc_mesh = plsc.VectorSubcoreMesh(
    core_axis_name="c", subcore_axis_name="s", num_cores=NC, num_subcores=NS
)


@functools.partial(
    pl.kernel,
    out_type=jax.ShapeDtypeStruct((TILES_TOT, C_PAD, 128), jnp.float32),
    mesh=_sc_mesh,
    scratch_types=[
        [pltpu.VMEM((CHUNK,), jnp.int32)] * NBUF,
        [pltpu.VMEM((CHUNK, ROW_W), jnp.float32)] * NBUF,
        [pltpu.VMEM((CHUNK // 128, C_PAD, 128), jnp.float32)] * NBUF,
        [pltpu.SemaphoreType.DMA] * NBUF,
        [pltpu.SemaphoreType.DMA] * NBUF,
    ],
    compiler_params=pltpu.CompilerParams(
        use_tc_tiling_on_sc=False, needs_layout_passes=False
    ),
)
def _sc_gather(table_hbm, idx_hbm, out_hbm, idx_vs, rows_vs, xt_vs, gsems, osems):
    wid = lax.axis_index("s") * NC + lax.axis_index("c")
    tile_base = wid * NT_BASE + jnp.minimum(wid, NT_XTRA)
    base = tile_base * 128
    lane = lax.iota(jnp.int32, 16)

    def _transpose(b, ntiles):
        rows = rows_vs[b]
        xt = xt_vs[b]

        @pl.loop(0, ntiles)
        def _tile(t):
            r0 = t * 128
            for c in range(N_COMM):
                cvec = jnp.full((16,), c, jnp.int32)
                for g in range(8):
                    ridx = r0 + g * 16 + lane
                    xt[t, c, pl.ds(g * 16, 16)] = plsc.load_gather(
                        rows, [ridx, cvec]
                    )

    def _gather_chunk(b, roff, n):
        pltpu.sync_copy(
            idx_hbm.at[pl.ds(base + roff, n)], idx_vs[b].at[pl.ds(0, n)]
        )
        return pltpu.async_copy(
            table_hbm.at[idx_vs[b].at[pl.ds(0, n)]],
            rows_vs[b].at[pl.ds(0, n)],
            gsems[b],
        )

    def _out_copy(i):
        roff, n, nt = CHUNK_PLAN[i]
        b = i % NBUF
        return pltpu.async_copy(
            xt_vs[b].at[pl.ds(0, nt)],
            out_hbm.at[pl.ds(tile_base + roff // 128, nt)],
            osems[b],
        )

    n_chunk = len(CHUNK_PLAN)
    gd = [None] * n_chunk
    od = [None] * n_chunk
    for i in range(n_chunk):
        b = i % NBUF
        if i >= NBUF:
            od[i - NBUF].wait()
        roff, n, nt = CHUNK_PLAN[i]
        gd[i] = _gather_chunk(b, roff, n)
        if i >= 1:
            gd[i - 1].wait()
            _transpose((i - 1) % NBUF, CHUNK_PLAN[i - 1][2])
            od[i - 1] = _out_copy(i - 1)
    gd[n_chunk - 1].wait()
    _transpose((n_chunk - 1) % NBUF, CHUNK_PLAN[n_chunk - 1][2])
    od[n_chunk - 1] = _out_copy(n_chunk - 1)
    for i in range(max(0, n_chunk - NBUF), n_chunk):
        od[i].wait()

    @pl.when(wid < NT_XTRA)
    def _extra():
        pltpu.sync_copy(
            idx_hbm.at[pl.ds(base + XTRA_OFF, 128)], idx_vs[0].at[pl.ds(0, 128)]
        )
        pltpu.async_copy(
            table_hbm.at[idx_vs[0].at[pl.ds(0, 128)]],
            rows_vs[0].at[pl.ds(0, 128)],
            gsems[0],
        ).wait()
        _transpose(0, 1)
        pltpu.sync_copy(
            xt_vs[0].at[pl.ds(0, 1)],
            out_hbm.at[pl.ds(tile_base + NT_BASE, 1)],
        )


def _fmt_body(x_ref, out_ref):
    for j in range(FMT_TILES):
        out_ref[:, j * 128:(j + 1) * 128] = x_ref[j, :N_COMM, :]


_fmt_call = pl.pallas_call(
    _fmt_body,
    grid=(FMT_GRID,),
    in_specs=[pl.BlockSpec((FMT_TILES, C_PAD, 128), lambda i: (i, 0, 0))],
    out_specs=pl.BlockSpec((N_COMM, FMT_BLK), lambda i: (0, i)),
    out_shape=jax.ShapeDtypeStruct((N_COMM, BATCH), jnp.float32),
)


def kernel(node_features, nodes, W1, b1, W2, b2):
    table3 = _mlp_call(
        node_features, W1, b1.reshape(1, D_FEAT), W2, b2.reshape(1, N_COMM)
    )
    table = table3.reshape(N_NODES * 8, ROW_W)
    nodes_p = jnp.zeros((B_EFF,), jnp.int32).at[:BATCH].set(nodes * 8)
    tiles = _sc_gather(table, nodes_p)
    out_t = _fmt_call(tiles)
    return out_t.T


# speedup vs baseline: 8.6065x; 1.0522x over previous
import functools

import jax
import jax.numpy as jnp
from jax import lax
from jax.experimental import pallas as pl
from jax.experimental.pallas import tpu as pltpu
from jax.experimental.pallas import tpu_sc as plsc

N_NODES = 100000
D_FEAT = 128
N_COMM = 5
BATCH = 500000

C_PAD = 8
ROW_BLK = 10000

NC, NS = 2, 16
NW = NC * NS
TILES_TOT = -(-BATCH // 128)
B_EFF = TILES_TOT * 128
NT_BASE = TILES_TOT // NW
NT_XTRA = TILES_TOT % NW
CHUNK = 2048
NBUF = 2
ROW_W = 16
CHUNK_PLAN = [(k * CHUNK, CHUNK, CHUNK // 128) for k in range(7)]
CHUNK_PLAN.append((7 * CHUNK, NT_BASE * 128 - 7 * CHUNK, NT_BASE - 7 * 16))
XTRA_OFF = NT_BASE * 128

FMT_TILES = 128
FMT_BLK = FMT_TILES * 128
FMT_GRID = -(-BATCH // FMT_BLK)


def _mlp_body(nf_ref, w1_ref, b1_ref, w2_ref, b2_ref, out_ref):
    x = nf_ref[...]
    h = jnp.dot(x, w1_ref[...], preferred_element_type=jnp.float32)
    h = jnp.maximum(h + b1_ref[...], 0.0)
    logits = jnp.dot(h, w2_ref[...], preferred_element_type=jnp.float32)
    logits = logits + b2_ref[...]
    m = jnp.max(logits, axis=-1, keepdims=True)
    e = jnp.exp(logits - m)
    p = e / jnp.sum(e, axis=-1, keepdims=True)
    out_ref[:, :, 0:N_COMM] = p.reshape(ROW_BLK // 8, 8, N_COMM)


_mlp_call = pl.pallas_call(
    _mlp_body,
    grid=(N_NODES // ROW_BLK,),
    in_specs=[
        pl.BlockSpec((ROW_BLK, D_FEAT), lambda i: (i, 0)),
        pl.BlockSpec((D_FEAT, D_FEAT), lambda i: (0, 0)),
        pl.BlockSpec((1, D_FEAT), lambda i: (0, 0)),
        pl.BlockSpec((D_FEAT, N_COMM), lambda i: (0, 0)),
        pl.BlockSpec((1, N_COMM), lambda i: (0, 0)),
    ],
    out_specs=pl.BlockSpec((ROW_BLK // 8, 8, 128), lambda i: (i, 0, 0)),
    out_shape=jax.ShapeDtypeStruct((N_NODES // 8, 8, 128), jnp.float32),
)

_sc_mesh = plsc.VectorSubcoreMesh(
    core_axis_name="c", subcore_axis_name="s", num_cores=NC, num_subcores=NS
)


@functools.partial(
    pl.kernel,
    out_type=jax.ShapeDtypeStruct((TILES_TOT, C_PAD, 128), jnp.float32),
    mesh=_sc_mesh,
    scratch_types=[
        [pltpu.VMEM((CHUNK,), jnp.int32)] * NBUF,
        [pltpu.VMEM((CHUNK, ROW_W), jnp.float32)] * NBUF,
        [pltpu.VMEM((CHUNK // 128, C_PAD, 128), jnp.float32)] * NBUF,
        [pltpu.SemaphoreType.DMA] * NBUF,
        [pltpu.SemaphoreType.DMA] * NBUF,
    ],
    compiler_params=pltpu.CompilerParams(
        use_tc_tiling_on_sc=False, needs_layout_passes=False
    ),
)
def _sc_gather(table_hbm, idx_hbm, out_hbm, idx_vs, rows_vs, xt_vs, gsems, osems):
    wid = lax.axis_index("s") * NC + lax.axis_index("c")
    tile_base = wid * NT_BASE + jnp.minimum(wid, NT_XTRA)
    base = tile_base * 128
    lane = lax.iota(jnp.int32, 16)

    def _transpose(b, ntiles):
        rows = rows_vs[b]
        xt = xt_vs[b]

        @pl.loop(0, ntiles)
        def _tile(t):
            r0 = t * 128
            for c in range(N_COMM):
                cvec = jnp.full((16,), c, jnp.int32)
                for g in range(8):
                    ridx = r0 + g * 16 + lane
                    xt[t, c, pl.ds(g * 16, 16)] = plsc.load_gather(
                        rows, [ridx, cvec]
                    )

    def _gather_chunk(b, roff, n):
        pltpu.sync_copy(
            idx_hbm.at[pl.ds(base + roff, n)], idx_vs[b].at[pl.ds(0, n)]
        )
        return pltpu.async_copy(
            table_hbm.at[idx_vs[b].at[pl.ds(0, n)]],
            rows_vs[b].at[pl.ds(0, n)],
            gsems[b],
        )

    def _out_copy(i):
        roff, n, nt = CHUNK_PLAN[i]
        b = i % NBUF
        return pltpu.async_copy(
            xt_vs[b].at[pl.ds(0, nt)],
            out_hbm.at[pl.ds(tile_base + roff // 128, nt)],
            osems[b],
        )

    n_chunk = len(CHUNK_PLAN)
    gd = [None] * n_chunk
    od = [None] * n_chunk
    for i in range(n_chunk):
        b = i % NBUF
        if i >= NBUF:
            od[i - NBUF].wait()
        roff, n, nt = CHUNK_PLAN[i]
        gd[i] = _gather_chunk(b, roff, n)
        if i >= 1:
            gd[i - 1].wait()
            _transpose((i - 1) % NBUF, CHUNK_PLAN[i - 1][2])
            od[i - 1] = _out_copy(i - 1)
    gd[n_chunk - 1].wait()
    _transpose((n_chunk - 1) % NBUF, CHUNK_PLAN[n_chunk - 1][2])
    od[n_chunk - 1] = _out_copy(n_chunk - 1)
    for i in range(max(0, n_chunk - NBUF), n_chunk):
        od[i].wait()

    @pl.when(wid < NT_XTRA)
    def _extra():
        pltpu.sync_copy(
            idx_hbm.at[pl.ds(base + XTRA_OFF, 128)], idx_vs[0].at[pl.ds(0, 128)]
        )
        pltpu.async_copy(
            table_hbm.at[idx_vs[0].at[pl.ds(0, 128)]],
            rows_vs[0].at[pl.ds(0, 128)],
            gsems[0],
        ).wait()
        _transpose(0, 1)
        pltpu.sync_copy(
            xt_vs[0].at[pl.ds(0, 1)],
            out_hbm.at[pl.ds(tile_base + NT_BASE, 1)],
        )


def _fmt_body(x_ref, out_ref):
    for j in range(FMT_TILES):
        out_ref[:, j * 128:(j + 1) * 128] = x_ref[j, :N_COMM, :]


_fmt_call = pl.pallas_call(
    _fmt_body,
    grid=(FMT_GRID,),
    in_specs=[pl.BlockSpec((FMT_TILES, C_PAD, 128), lambda i: (i, 0, 0))],
    out_specs=pl.BlockSpec((N_COMM, FMT_BLK), lambda i: (0, i)),
    out_shape=jax.ShapeDtypeStruct((N_COMM, BATCH), jnp.float32),
)


def kernel(node_features, nodes, W1, b1, W2, b2):
    table3 = _mlp_call(
        node_features, W1, b1.reshape(1, D_FEAT), W2, b2.reshape(1, N_COMM)
    )
    table = table3.reshape(N_NODES * 8, ROW_W)
    nodes_p = jnp.zeros((B_EFF,), jnp.int32).at[:BATCH].set(nodes * 8)
    tiles = _sc_gather(table, nodes_p)
    out_t = _fmt_call(tiles)
    return out_t.T


# speedup vs baseline: 9.0153x; 1.0475x over previous
import functools

import jax
import jax.numpy as jnp
from jax import lax
from jax.experimental import pallas as pl
from jax.experimental.pallas import tpu as pltpu
from jax.experimental.pallas import tpu_sc as plsc

N_NODES = 100000
D_FEAT = 128
N_COMM = 5
BATCH = 500000

C_PAD = 8
ROW_BLK = 20000

NC, NS = 2, 16
NW = NC * NS
TILES_TOT = -(-BATCH // 128)
B_EFF = TILES_TOT * 128
NT_BASE = TILES_TOT // NW
NT_XTRA = TILES_TOT % NW
CHUNK = 2048
NBUF = 2
ROW_W = 16
CHUNK_PLAN = [(k * CHUNK, CHUNK, CHUNK // 128) for k in range(7)]
CHUNK_PLAN.append((7 * CHUNK, NT_BASE * 128 - 7 * CHUNK, NT_BASE - 7 * 16))
XTRA_OFF = NT_BASE * 128

FMT_TILES = 256
FMT_BLK = FMT_TILES * 128
FMT_GRID = -(-BATCH // FMT_BLK)


def _mlp_body(nf_ref, w1_ref, b1_ref, w2_ref, b2_ref, out_ref):
    x = nf_ref[...]
    h = jnp.dot(x, w1_ref[...], preferred_element_type=jnp.float32)
    h = jnp.maximum(h + b1_ref[...], 0.0)
    logits = jnp.dot(h, w2_ref[...], preferred_element_type=jnp.float32)
    logits = logits + b2_ref[...]
    m = jnp.max(logits, axis=-1, keepdims=True)
    e = jnp.exp(logits - m)
    p = e / jnp.sum(e, axis=-1, keepdims=True)
    out_ref[:, :, 0:N_COMM] = p.reshape(ROW_BLK // 8, 8, N_COMM)


_mlp_call = pl.pallas_call(
    _mlp_body,
    grid=(N_NODES // ROW_BLK,),
    in_specs=[
        pl.BlockSpec((ROW_BLK, D_FEAT), lambda i: (i, 0)),
        pl.BlockSpec((D_FEAT, D_FEAT), lambda i: (0, 0)),
        pl.BlockSpec((1, D_FEAT), lambda i: (0, 0)),
        pl.BlockSpec((D_FEAT, N_COMM), lambda i: (0, 0)),
        pl.BlockSpec((1, N_COMM), lambda i: (0, 0)),
    ],
    out_specs=pl.BlockSpec((ROW_BLK // 8, 8, 128), lambda i: (i, 0, 0)),
    out_shape=jax.ShapeDtypeStruct((N_NODES // 8, 8, 128), jnp.float32),
)

_sc_mesh = plsc.VectorSubcoreMesh(
    core_axis_name="c", subcore_axis_name="s", num_cores=NC, num_subcores=NS
)


@functools.partial(
    pl.kernel,
    out_type=jax.ShapeDtypeStruct((TILES_TOT, C_PAD, 128), jnp.float32),
    mesh=_sc_mesh,
    scratch_types=[
        [pltpu.VMEM((CHUNK,), jnp.int32)] * NBUF,
        [pltpu.VMEM((CHUNK, ROW_W), jnp.float32)] * NBUF,
        [pltpu.VMEM((CHUNK // 128, C_PAD, 128), jnp.float32)] * NBUF,
        [pltpu.SemaphoreType.DMA] * NBUF,
        [pltpu.SemaphoreType.DMA] * NBUF,
    ],
    compiler_params=pltpu.CompilerParams(
        use_tc_tiling_on_sc=False, needs_layout_passes=False
    ),
)
def _sc_gather(table_hbm, idx_hbm, out_hbm, idx_vs, rows_vs, xt_vs, gsems, osems):
    wid = lax.axis_index("s") * NC + lax.axis_index("c")
    tile_base = wid * NT_BASE + jnp.minimum(wid, NT_XTRA)
    base = tile_base * 128
    lane = lax.iota(jnp.int32, 16)

    def _transpose(b, ntiles):
        rows = rows_vs[b]
        xt = xt_vs[b]

        @pl.loop(0, ntiles)
        def _tile(t):
            r0 = t * 128
            for c in range(N_COMM):
                cvec = jnp.full((16,), c, jnp.int32)
                for g in range(8):
                    ridx = r0 + g * 16 + lane
                    xt[t, c, pl.ds(g * 16, 16)] = plsc.load_gather(
                        rows, [ridx, cvec]
                    )

    def _gather_chunk(b, roff, n):
        pltpu.sync_copy(
            idx_hbm.at[pl.ds(base + roff, n)], idx_vs[b].at[pl.ds(0, n)]
        )
        return pltpu.async_copy(
            table_hbm.at[idx_vs[b].at[pl.ds(0, n)]],
            rows_vs[b].at[pl.ds(0, n)],
            gsems[b],
        )

    def _out_copy(i):
        roff, n, nt = CHUNK_PLAN[i]
        b = i % NBUF
        return pltpu.async_copy(
            xt_vs[b].at[pl.ds(0, nt)],
            out_hbm.at[pl.ds(tile_base + roff // 128, nt)],
            osems[b],
        )

    n_chunk = len(CHUNK_PLAN)
    gd = [None] * n_chunk
    od = [None] * n_chunk
    for i in range(n_chunk):
        b = i % NBUF
        if i >= NBUF:
            od[i - NBUF].wait()
        roff, n, nt = CHUNK_PLAN[i]
        gd[i] = _gather_chunk(b, roff, n)
        if i >= 1:
            gd[i - 1].wait()
            _transpose((i - 1) % NBUF, CHUNK_PLAN[i - 1][2])
            od[i - 1] = _out_copy(i - 1)
    gd[n_chunk - 1].wait()
    _transpose((n_chunk - 1) % NBUF, CHUNK_PLAN[n_chunk - 1][2])
    od[n_chunk - 1] = _out_copy(n_chunk - 1)
    for i in range(max(0, n_chunk - NBUF), n_chunk):
        od[i].wait()

    @pl.when(wid < NT_XTRA)
    def _extra():
        pltpu.sync_copy(
            idx_hbm.at[pl.ds(base + XTRA_OFF, 128)], idx_vs[0].at[pl.ds(0, 128)]
        )
        pltpu.async_copy(
            table_hbm.at[idx_vs[0].at[pl.ds(0, 128)]],
            rows_vs[0].at[pl.ds(0, 128)],
            gsems[0],
        ).wait()
        _transpose(0, 1)
        pltpu.sync_copy(
            xt_vs[0].at[pl.ds(0, 1)],
            out_hbm.at[pl.ds(tile_base + NT_BASE, 1)],
        )


def _fmt_body(x_ref, out_ref):
    for j in range(FMT_TILES):
        out_ref[:, j * 128:(j + 1) * 128] = x_ref[j, :N_COMM, :]


_fmt_call = pl.pallas_call(
    _fmt_body,
    grid=(FMT_GRID,),
    in_specs=[pl.BlockSpec((FMT_TILES, C_PAD, 128), lambda i: (i, 0, 0))],
    out_specs=pl.BlockSpec((N_COMM, FMT_BLK), lambda i: (0, i)),
    out_shape=jax.ShapeDtypeStruct((N_COMM, BATCH), jnp.float32),
)


def kernel(node_features, nodes, W1, b1, W2, b2):
    table3 = _mlp_call(
        node_features, W1, b1.reshape(1, D_FEAT), W2, b2.reshape(1, N_COMM)
    )
    table = table3.reshape(N_NODES * 8, ROW_W)
    nodes_p = jnp.zeros((B_EFF,), jnp.int32).at[:BATCH].set(nodes * 8)
    tiles = _sc_gather(table, nodes_p)
    out_t = _fmt_call(tiles)
    return out_t.T


# speedup vs baseline: 9.3390x; 1.0359x over previous
import functools

import jax
import jax.numpy as jnp
from jax import lax
from jax.experimental import pallas as pl
from jax.experimental.pallas import tpu as pltpu
from jax.experimental.pallas import tpu_sc as plsc

N_NODES = 100000
D_FEAT = 128
N_COMM = 5
BATCH = 500000

C_PAD = 8
ROW_BLK = 25000

NC, NS = 2, 16
NW = NC * NS
TILES_TOT = -(-BATCH // 128)
B_EFF = TILES_TOT * 128
NT_BASE = TILES_TOT // NW
NT_XTRA = TILES_TOT % NW
CHUNK = 2048
NBUF = 2
ROW_W = 16
CHUNK_PLAN = [(k * CHUNK, CHUNK, CHUNK // 128) for k in range(7)]
CHUNK_PLAN.append((7 * CHUNK, NT_BASE * 128 - 7 * CHUNK, NT_BASE - 7 * 16))
XTRA_OFF = NT_BASE * 128

FMT_TILES = 512
FMT_BLK = FMT_TILES * 128
FMT_GRID = -(-BATCH // FMT_BLK)


def _mlp_body(nf_ref, w1_ref, b1_ref, w2_ref, b2_ref, out_ref):
    x = nf_ref[...]
    h = jnp.dot(x, w1_ref[...], preferred_element_type=jnp.float32)
    h = jnp.maximum(h + b1_ref[...], 0.0)
    logits = jnp.dot(h, w2_ref[...], preferred_element_type=jnp.float32)
    logits = logits + b2_ref[...]
    m = jnp.max(logits, axis=-1, keepdims=True)
    e = jnp.exp(logits - m)
    p = e / jnp.sum(e, axis=-1, keepdims=True)
    out_ref[:, :, 0:N_COMM] = p.reshape(ROW_BLK // 8, 8, N_COMM)


_mlp_call = pl.pallas_call(
    _mlp_body,
    grid=(N_NODES // ROW_BLK,),
    in_specs=[
        pl.BlockSpec((ROW_BLK, D_FEAT), lambda i: (i, 0)),
        pl.BlockSpec((D_FEAT, D_FEAT), lambda i: (0, 0)),
        pl.BlockSpec((1, D_FEAT), lambda i: (0, 0)),
        pl.BlockSpec((D_FEAT, N_COMM), lambda i: (0, 0)),
        pl.BlockSpec((1, N_COMM), lambda i: (0, 0)),
    ],
    out_specs=pl.BlockSpec((ROW_BLK // 8, 8, 128), lambda i: (i, 0, 0)),
    out_shape=jax.ShapeDtypeStruct((N_NODES // 8, 8, 128), jnp.float32),
)

_sc_mesh = plsc.VectorSubcoreMesh(
    core_axis_name="c", subcore_axis_name="s", num_cores=NC, num_subcores=NS
)


@functools.partial(
    pl.kernel,
    out_type=jax.ShapeDtypeStruct((TILES_TOT, C_PAD, 128), jnp.float32),
    mesh=_sc_mesh,
    scratch_types=[
        [pltpu.VMEM((CHUNK,), jnp.int32)] * NBUF,
        [pltpu.VMEM((CHUNK, ROW_W), jnp.float32)] * NBUF,
        [pltpu.VMEM((CHUNK // 128, C_PAD, 128), jnp.float32)] * NBUF,
        [pltpu.SemaphoreType.DMA] * NBUF,
        [pltpu.SemaphoreType.DMA] * NBUF,
    ],
    compiler_params=pltpu.CompilerParams(
        use_tc_tiling_on_sc=False, needs_layout_passes=False
    ),
)
def _sc_gather(table_hbm, idx_hbm, out_hbm, idx_vs, rows_vs, xt_vs, gsems, osems):
    wid = lax.axis_index("s") * NC + lax.axis_index("c")
    tile_base = wid * NT_BASE + jnp.minimum(wid, NT_XTRA)
    base = tile_base * 128
    lane = lax.iota(jnp.int32, 16)

    def _transpose(b, ntiles):
        rows = rows_vs[b]
        xt = xt_vs[b]

        @pl.loop(0, ntiles)
        def _tile(t):
            r0 = t * 128
            for c in range(N_COMM):
                cvec = jnp.full((16,), c, jnp.int32)
                for g in range(8):
                    ridx = r0 + g * 16 + lane
                    xt[t, c, pl.ds(g * 16, 16)] = plsc.load_gather(
                        rows, [ridx, cvec]
                    )

    def _gather_chunk(b, roff, n):
        pltpu.sync_copy(
            idx_hbm.at[pl.ds(base + roff, n)], idx_vs[b].at[pl.ds(0, n)]
        )
        return pltpu.async_copy(
            table_hbm.at[idx_vs[b].at[pl.ds(0, n)]],
            rows_vs[b].at[pl.ds(0, n)],
            gsems[b],
        )

    def _out_copy(i):
        roff, n, nt = CHUNK_PLAN[i]
        b = i % NBUF
        return pltpu.async_copy(
            xt_vs[b].at[pl.ds(0, nt)],
            out_hbm.at[pl.ds(tile_base + roff // 128, nt)],
            osems[b],
        )

    n_chunk = len(CHUNK_PLAN)
    gd = [None] * n_chunk
    od = [None] * n_chunk
    for i in range(n_chunk):
        b = i % NBUF
        if i >= NBUF:
            od[i - NBUF].wait()
        roff, n, nt = CHUNK_PLAN[i]
        gd[i] = _gather_chunk(b, roff, n)
        if i >= 1:
            gd[i - 1].wait()
            _transpose((i - 1) % NBUF, CHUNK_PLAN[i - 1][2])
            od[i - 1] = _out_copy(i - 1)
    gd[n_chunk - 1].wait()
    _transpose((n_chunk - 1) % NBUF, CHUNK_PLAN[n_chunk - 1][2])
    od[n_chunk - 1] = _out_copy(n_chunk - 1)
    for i in range(max(0, n_chunk - NBUF), n_chunk):
        od[i].wait()

    @pl.when(wid < NT_XTRA)
    def _extra():
        pltpu.sync_copy(
            idx_hbm.at[pl.ds(base + XTRA_OFF, 128)], idx_vs[0].at[pl.ds(0, 128)]
        )
        pltpu.async_copy(
            table_hbm.at[idx_vs[0].at[pl.ds(0, 128)]],
            rows_vs[0].at[pl.ds(0, 128)],
            gsems[0],
        ).wait()
        _transpose(0, 1)
        pltpu.sync_copy(
            xt_vs[0].at[pl.ds(0, 1)],
            out_hbm.at[pl.ds(tile_base + NT_BASE, 1)],
        )


def _fmt_body(x_ref, out_ref):
    for j in range(FMT_TILES):
        out_ref[:, j * 128:(j + 1) * 128] = x_ref[j, :N_COMM, :]


_fmt_call = pl.pallas_call(
    _fmt_body,
    grid=(FMT_GRID,),
    in_specs=[pl.BlockSpec((FMT_TILES, C_PAD, 128), lambda i: (i, 0, 0))],
    out_specs=pl.BlockSpec((N_COMM, FMT_BLK), lambda i: (0, i)),
    out_shape=jax.ShapeDtypeStruct((N_COMM, BATCH), jnp.float32),
)


def kernel(node_features, nodes, W1, b1, W2, b2):
    table3 = _mlp_call(
        node_features, W1, b1.reshape(1, D_FEAT), W2, b2.reshape(1, N_COMM)
    )
    table = table3.reshape(N_NODES * 8, ROW_W)
    nodes_p = jnp.zeros((B_EFF,), jnp.int32).at[:BATCH].set(nodes * 8)
    tiles = _sc_gather(table, nodes_p)
    out_t = _fmt_call(tiles)
    return out_t.T
